# baseline (speedup 1.0000x reference)
import functools

import jax
import jax.numpy as jnp
from jax import lax
from jax.experimental import pallas as pl
from jax.experimental.pallas import tpu as pltpu

_PAR = pltpu.CompilerParams(dimension_semantics=("parallel",))

N = 2048
H = 6
BLK = 256
PBLK = 512


def _mask_kernel(adj_ref, m_ref):
    m_ref[...] = (adj_ref[...] > 0).astype(jnp.int8)


def _proj_kernel(x_ref, w_ref, h_ref):
    h_ref[...] = jnp.dot(x_ref[...], w_ref[...],
                         preferred_element_type=jnp.float32)


def _b16(v):
    return v.astype(jnp.bfloat16).astype(jnp.float32)


def _attn_kernel(F, h_ref, hb_ref, m_ref, a_ref, o_ref):
    bias = (m_ref[...].astype(jnp.float32) - 1.0) * 1e9
    hb_all = hb_ref[...]
    for h in range(H):
        hv = h_ref[:, h * F:(h + 1) * F]
        hb = hb_all[:, h * F:(h + 1) * F]
        asrc = a_ref[h:h + 1, :F]
        adst = a_ref[h:h + 1, F:2 * F]
        if F == 16:
            es = jnp.sum(_b16(hb) * _b16(asrc), axis=1, keepdims=True)
        else:
            es = jnp.sum(hb * asrc, axis=1, keepdims=True)
        ed = lax.dot_general(adst, hv, (((1,), (1,)), ((), ())),
                             preferred_element_type=jnp.float32)
        e = es + ed
        e = jnp.maximum(e, 0.2 * e) + bias
        mx = jnp.max(e, axis=1, keepdims=True)
        p = jnp.exp(e - mx)
        s = jnp.sum(p, axis=1, keepdims=True)
        alpha = p / s
        out = jnp.dot(alpha, hv, preferred_element_type=jnp.float32)
        out = jnp.where(out > 0, out, jnp.exp(jnp.minimum(out, 0.0)) - 1.0)
        o_ref[:, h * F:(h + 1) * F] = out


def _head_kernel(h_ref, wdt_ref, bd_ref, o_ref):
    g = jnp.sum(h_ref[...], axis=0, keepdims=True)
    nrm = jnp.maximum(jnp.sqrt(jnp.sum(g * g)), 1e-12)
    g = g / nrm
    o_ref[...] = (jnp.sum(g * wdt_ref[...], axis=1, keepdims=True)
                  + bd_ref[...])


def _gat_layer(x, mask8, W, a):
    Hh, Din, F = W.shape
    w_flat = jnp.transpose(W, (1, 0, 2)).reshape(Din, Hh * F)
    if Din % 8:
        pad = 8 - Din % 8
        x = jnp.pad(x, ((0, 0), (0, pad)))
        w_flat = jnp.pad(w_flat, ((0, pad), (0, 0)))
        Din += pad
    h_all = pl.pallas_call(
        _proj_kernel,
        grid=(N // PBLK,),
        in_specs=[
            pl.BlockSpec((PBLK, Din), lambda i: (i, 0)),
            pl.BlockSpec((Din, Hh * F), lambda i: (0, 0)),
        ],
        out_specs=pl.BlockSpec((PBLK, Hh * F), lambda i: (i, 0)),
        out_shape=jax.ShapeDtypeStruct((N, Hh * F), jnp.float32),
    )(x, w_flat)
    out = pl.pallas_call(
        functools.partial(_attn_kernel, F),
        grid=(N // BLK,),
        in_specs=[
            pl.BlockSpec((N, Hh * F), lambda i: (0, 0)),
            pl.BlockSpec((BLK, Hh * F), lambda i: (i, 0)),
            pl.BlockSpec((BLK, N), lambda i: (i, 0)),
            pl.BlockSpec((Hh, 2 * F), lambda i: (0, 0)),
        ],
        out_specs=pl.BlockSpec((BLK, Hh * F), lambda i: (i, 0)),
        out_shape=jax.ShapeDtypeStruct((N, Hh * F), jnp.float32),
        compiler_params=_PAR,
    )(h_all, h_all, mask8, a)
    return out


def kernel(x, adj, W1, a1, W2, a2, W3, a3, Wd, bd):
    mask8 = pl.pallas_call(
        _mask_kernel,
        grid=(N // BLK,),
        in_specs=[pl.BlockSpec((BLK, N), lambda i: (i, 0))],
        out_specs=pl.BlockSpec((BLK, N), lambda i: (i, 0)),
        out_shape=jax.ShapeDtypeStruct((N, N), jnp.int8),
    )(adj)
    h = _gat_layer(x, mask8, W1, a1)
    h = _gat_layer(h, mask8, W2, a2)
    h = _gat_layer(h, mask8, W3, a3)
    out = pl.pallas_call(
        _head_kernel,
        in_specs=[
            pl.BlockSpec((N, 384), lambda: (0, 0)),
            pl.BlockSpec((1, 384), lambda: (0, 0)),
            pl.BlockSpec((1, 1), lambda: (0, 0)),
        ],
        out_specs=pl.BlockSpec((1, 1), lambda: (0, 0)),
        out_shape=jax.ShapeDtypeStruct((1, 1), jnp.float32),
    )(h, Wd.reshape(1, 384), bd.reshape(1, 1))
    return out.reshape(1)


# speedup vs baseline: 1.0776x; 1.0776x over previous
import functools

import jax
import jax.numpy as jnp
from jax import lax
from jax.experimental import pallas as pl
from jax.experimental.pallas import tpu as pltpu

_PAR = pltpu.CompilerParams(dimension_semantics=("parallel",))

N = 2048
H = 6
BLK = 512
PBLK = 512


def _mask_kernel(adj_ref, m_ref):
    m_ref[...] = (adj_ref[...] > 0).astype(jnp.int8)


def _proj_kernel(x_ref, w_ref, h_ref):
    h_ref[...] = jnp.dot(x_ref[...], w_ref[...],
                         preferred_element_type=jnp.float32)


def _b16(v):
    return v.astype(jnp.bfloat16).astype(jnp.float32)


def _attn_kernel(F, h_ref, m_ref, a_ref, o_ref):
    i = pl.program_id(0)
    bias = (m_ref[...].astype(jnp.float32) - 1.0) * 1e9
    hb_all = h_ref[pl.ds(i * BLK, BLK), :]
    for h in range(H):
        hv = h_ref[:, h * F:(h + 1) * F]
        hb = hb_all[:, h * F:(h + 1) * F]
        asrc = a_ref[h:h + 1, :F]
        adst = a_ref[h:h + 1, F:2 * F]
        if F == 16:
            es = jnp.sum(_b16(hb) * _b16(asrc), axis=1, keepdims=True)
        else:
            es = jnp.sum(hb * asrc, axis=1, keepdims=True)
        ed = lax.dot_general(adst, hv, (((1,), (1,)), ((), ())),
                             preferred_element_type=jnp.float32)
        e = es + ed
        e = jnp.maximum(e, 0.2 * e) + bias
        mx = jnp.max(e, axis=1, keepdims=True)
        p = jnp.exp(e - mx)
        s = jnp.sum(p, axis=1, keepdims=True)
        alpha = p / s
        out = jnp.dot(alpha, hv, preferred_element_type=jnp.float32)
        out = jnp.where(out > 0, out, jnp.exp(jnp.minimum(out, 0.0)) - 1.0)
        o_ref[:, h * F:(h + 1) * F] = out


def _head_kernel(h_ref, wdt_ref, bd_ref, o_ref):
    g = jnp.sum(h_ref[...], axis=0, keepdims=True)
    nrm = jnp.maximum(jnp.sqrt(jnp.sum(g * g)), 1e-12)
    g = g / nrm
    o_ref[...] = (jnp.sum(g * wdt_ref[...], axis=1, keepdims=True)
                  + bd_ref[...])


def _gat_layer(x, mask8, W, a):
    Hh, Din, F = W.shape
    w_flat = jnp.transpose(W, (1, 0, 2)).reshape(Din, Hh * F)
    if Din % 8:
        pad = 8 - Din % 8
        x = jnp.pad(x, ((0, 0), (0, pad)))
        w_flat = jnp.pad(w_flat, ((0, pad), (0, 0)))
        Din += pad
    h_all = pl.pallas_call(
        _proj_kernel,
        grid=(N // PBLK,),
        in_specs=[
            pl.BlockSpec((PBLK, Din), lambda i: (i, 0)),
            pl.BlockSpec((Din, Hh * F), lambda i: (0, 0)),
        ],
        out_specs=pl.BlockSpec((PBLK, Hh * F), lambda i: (i, 0)),
        out_shape=jax.ShapeDtypeStruct((N, Hh * F), jnp.float32),
    )(x, w_flat)
    out = pl.pallas_call(
        functools.partial(_attn_kernel, F),
        grid=(N // BLK,),
        in_specs=[
            pl.BlockSpec((N, Hh * F), lambda i: (0, 0)),
            pl.BlockSpec((BLK, N), lambda i: (i, 0)),
            pl.BlockSpec((Hh, 2 * F), lambda i: (0, 0)),
        ],
        out_specs=pl.BlockSpec((BLK, Hh * F), lambda i: (i, 0)),
        out_shape=jax.ShapeDtypeStruct((N, Hh * F), jnp.float32),
        compiler_params=_PAR,
    )(h_all, mask8, a)
    return out


def kernel(x, adj, W1, a1, W2, a2, W3, a3, Wd, bd):
    mask8 = pl.pallas_call(
        _mask_kernel,
        grid=(N // BLK,),
        in_specs=[pl.BlockSpec((BLK, N), lambda i: (i, 0))],
        out_specs=pl.BlockSpec((BLK, N), lambda i: (i, 0)),
        out_shape=jax.ShapeDtypeStruct((N, N), jnp.int8),
    )(adj)
    h = _gat_layer(x, mask8, W1, a1)
    h = _gat_layer(h, mask8, W2, a2)
    h = _gat_layer(h, mask8, W3, a3)
    out = pl.pallas_call(
        _head_kernel,
        in_specs=[
            pl.BlockSpec((N, 384), lambda: (0, 0)),
            pl.BlockSpec((1, 384), lambda: (0, 0)),
            pl.BlockSpec((1, 1), lambda: (0, 0)),
        ],
        out_specs=pl.BlockSpec((1, 1), lambda: (0, 0)),
        out_shape=jax.ShapeDtypeStruct((1, 1), jnp.float32),
    )(h, Wd.reshape(1, 384), bd.reshape(1, 1))
    return out.reshape(1)
